# baseline (speedup 1.0000x reference)
import functools

import jax
import jax.numpy as jnp
from jax import lax
from jax.experimental import pallas as pl
from jax.experimental.pallas import tpu as pltpu
from jax.experimental.pallas import tpu_sc as plsc

WEIGHT_CENT = 0.001
TAU = 1.0
ETA = 2.0
LOG2E = 1.4426950408889634

TILE_N = 256


def _sc_gather(matrix, target):
    n_cls, feat = matrix.shape
    batch = target.shape[0]
    info = plsc.get_sparse_core_info()
    nw = info.num_cores * info.num_subcores
    b_per_w = batch // nw
    mesh = plsc.VectorSubcoreMesh(core_axis_name="c", subcore_axis_name="s")

    @functools.partial(
        pl.kernel,
        mesh=mesh,
        out_type=jax.ShapeDtypeStruct((batch, feat), jnp.float32),
        scratch_types=[
            pltpu.VMEM((b_per_w,), jnp.int32),
            pltpu.VMEM((b_per_w, feat), jnp.float32),
            pltpu.SemaphoreType.DMA,
        ],
    )
    def gather_kernel(table_hbm, idx_hbm, out_hbm, idx_v, rows_v, sem):
        wid = lax.axis_index("s") * info.num_cores + lax.axis_index("c")
        base = wid * b_per_w
        pltpu.sync_copy(idx_hbm.at[pl.ds(base, b_per_w)], idx_v)
        pltpu.async_copy(table_hbm.at[idx_v], rows_v, sem).wait()
        pltpu.sync_copy(rows_v, out_hbm.at[pl.ds(base, b_per_w)])

    return gather_kernel(matrix, target)


def _loss_kernel(m_ref, xf_ref, c_ref, out_ref, xn_s, cn_s, mn_s, sa_s, sb_s):
    j = pl.program_id(0)
    nj = pl.num_programs(0)
    batch = xf_ref.shape[0]
    dims = (((1,), (1,)), ((), ()))

    @pl.when(j == 0)
    def _init():
        x = xf_ref[...]
        c = c_ref[...]
        xn_s[...] = (x * lax.rsqrt(jnp.sum(x * x, axis=1, keepdims=True))).astype(
            jnp.bfloat16
        )
        cn_s[...] = (c * lax.rsqrt(jnp.sum(c * c, axis=1, keepdims=True))).astype(
            jnp.bfloat16
        )
        m = m_ref[...]
        mn_s[...] = (
            m * (lax.rsqrt(jnp.sum(m * m, axis=1, keepdims=True)) * (LOG2E / TAU))
        ).astype(jnp.bfloat16)
        icg = jnp.sum(jnp.square(x - c)) * (WEIGHT_CENT / batch)
        out_ref[...] = jnp.full((1, 1), icg, jnp.float32)

    def matmuls(sa_ref, sb_ref):
        mt = mn_s[pl.ds(j * TILE_N, TILE_N), :]
        sa_ref[...] = lax.dot_general(
            cn_s[...], mt, dims, preferred_element_type=jnp.float32
        )
        sb_ref[...] = lax.dot_general(
            xn_s[...], mt, dims, preferred_element_type=jnp.float32
        )

    def consume(sa_ref, sb_ref):
        d = jnp.exp2(sa_ref[...]) - jnp.exp2(sb_ref[...])
        part = jnp.sum(d * d) * (ETA / batch)
        out_ref[...] = out_ref[...] + jnp.full((1, 1), part, jnp.float32)

    even = j % 2 == 0

    @pl.when(jnp.logical_and(j < nj - 1, even))
    def _mm_even():
        matmuls(sa_s.at[0], sb_s.at[0])

    @pl.when(jnp.logical_and(j < nj - 1, jnp.logical_not(even)))
    def _mm_odd():
        matmuls(sa_s.at[1], sb_s.at[1])

    @pl.when(jnp.logical_and(j > 0, even))
    def _ew_even():
        consume(sa_s.at[1], sb_s.at[1])

    @pl.when(jnp.logical_and(j > 0, jnp.logical_not(even)))
    def _ew_odd():
        consume(sa_s.at[0], sb_s.at[0])


def _tc_loss(xf, centers, matrix):
    batch, feat = xf.shape
    n_cls = matrix.shape[0]
    n_tiles = n_cls // TILE_N
    out = pl.pallas_call(
        _loss_kernel,
        grid=(n_tiles + 1,),
        in_specs=[
            pl.BlockSpec((n_cls, feat), lambda j: (0, 0)),
            pl.BlockSpec((batch, feat), lambda j: (0, 0)),
            pl.BlockSpec((batch, feat), lambda j: (0, 0)),
        ],
        out_specs=pl.BlockSpec((1, 1), lambda j: (0, 0)),
        out_shape=jax.ShapeDtypeStruct((1, 1), jnp.float32),
        scratch_shapes=[
            pltpu.VMEM((batch, feat), jnp.bfloat16),
            pltpu.VMEM((batch, feat), jnp.bfloat16),
            pltpu.VMEM((n_cls, feat), jnp.bfloat16),
            pltpu.VMEM((2, batch, TILE_N), jnp.float32),
            pltpu.VMEM((2, batch, TILE_N), jnp.float32),
        ],
    )(matrix, xf, centers)
    return out[0, 0]


def kernel(xf, target, epoch, matrix):
    centers = _sc_gather(matrix, target.astype(jnp.int32))
    return _tc_loss(xf, centers, matrix)


# speedup vs baseline: 1.3478x; 1.3478x over previous
import functools

import jax
import jax.numpy as jnp
from jax import lax
from jax.experimental import pallas as pl
from jax.experimental.pallas import tpu as pltpu
from jax.experimental.pallas import tpu_sc as plsc

WEIGHT_CENT = 0.001
TAU = 1.0
ETA = 2.0
LOG2E = 1.4426950408889634

TILE_N = 256


def _sc_gather(matrix, target):
    n_cls, feat = matrix.shape
    batch = target.shape[0]
    info = plsc.get_sparse_core_info()
    nw = info.num_cores * info.num_subcores
    b_per_w = batch // nw
    mesh = plsc.VectorSubcoreMesh(core_axis_name="c", subcore_axis_name="s")

    @functools.partial(
        pl.kernel,
        mesh=mesh,
        out_type=jax.ShapeDtypeStruct((batch, feat), jnp.float32),
        scratch_types=[
            pltpu.VMEM((b_per_w,), jnp.int32),
            pltpu.VMEM((b_per_w, feat), jnp.float32),
            pltpu.SemaphoreType.DMA,
        ],
    )
    def gather_kernel(table_hbm, idx_hbm, out_hbm, idx_v, rows_v, sem):
        wid = lax.axis_index("s") * info.num_cores + lax.axis_index("c")
        base = wid * b_per_w
        pltpu.sync_copy(idx_hbm.at[pl.ds(base, b_per_w)], idx_v)
        pltpu.async_copy(table_hbm.at[idx_v], rows_v, sem).wait()
        pltpu.sync_copy(rows_v, out_hbm.at[pl.ds(base, b_per_w)])

    return gather_kernel(matrix, target)


def _loss_kernel(m_ref, xf_ref, c_ref, out_ref, xn_s, cn_s):
    j = pl.program_id(0)
    batch = xf_ref.shape[0]

    @pl.when(j == 0)
    def _init():
        x = xf_ref[...]
        c = c_ref[...]
        xn_s[...] = (x * lax.rsqrt(jnp.sum(x * x, axis=1, keepdims=True))).astype(
            jnp.bfloat16
        )
        cn_s[...] = (c * lax.rsqrt(jnp.sum(c * c, axis=1, keepdims=True))).astype(
            jnp.bfloat16
        )
        icg = jnp.sum(jnp.square(x - c)) * (WEIGHT_CENT / batch)
        out_ref[...] = jnp.full((1, 1), icg, jnp.float32)

    m = m_ref[...]
    mn = (
        m * (lax.rsqrt(jnp.sum(m * m, axis=1, keepdims=True)) * (LOG2E / TAU))
    ).astype(jnp.bfloat16)
    dims = (((1,), (1,)), ((), ()))
    sa = lax.dot_general(cn_s[...], mn, dims, preferred_element_type=jnp.float32)
    sb = lax.dot_general(xn_s[...], mn, dims, preferred_element_type=jnp.float32)
    d = jnp.exp2(sa) - jnp.exp2(sb)
    part = jnp.sum(d * d) * (ETA / batch)
    out_ref[...] = out_ref[...] + jnp.full((1, 1), part, jnp.float32)


def _tc_loss(xf, centers, matrix):
    batch, feat = xf.shape
    n_cls = matrix.shape[0]
    grid = (n_cls // TILE_N,)
    out = pl.pallas_call(
        _loss_kernel,
        grid=grid,
        in_specs=[
            pl.BlockSpec((TILE_N, feat), lambda j: (j, 0)),
            pl.BlockSpec((batch, feat), lambda j: (0, 0)),
            pl.BlockSpec((batch, feat), lambda j: (0, 0)),
        ],
        out_specs=pl.BlockSpec((1, 1), lambda j: (0, 0)),
        out_shape=jax.ShapeDtypeStruct((1, 1), jnp.float32),
        scratch_shapes=[
            pltpu.VMEM((batch, feat), jnp.bfloat16),
            pltpu.VMEM((batch, feat), jnp.bfloat16),
        ],
    )(matrix, xf, centers)
    return out[0, 0]


def kernel(xf, target, epoch, matrix):
    centers = _sc_gather(matrix, target.astype(jnp.int32))
    return _tc_loss(xf, centers, matrix)


# speedup vs baseline: 1.4476x; 1.0741x over previous
import functools

import jax
import jax.numpy as jnp
from jax import lax
from jax.experimental import pallas as pl
from jax.experimental.pallas import tpu as pltpu
from jax.experimental.pallas import tpu_sc as plsc

WEIGHT_CENT = 0.001
TAU = 1.0
ETA = 2.0
LOG2E = 1.4426950408889634

TILE_N = 512
SUB_N = 256


def _sc_gather(matrix, target):
    n_cls, feat = matrix.shape
    batch = target.shape[0]
    info = plsc.get_sparse_core_info()
    nw = info.num_cores * info.num_subcores
    b_per_w = batch // nw
    mesh = plsc.VectorSubcoreMesh(core_axis_name="c", subcore_axis_name="s")

    @functools.partial(
        pl.kernel,
        mesh=mesh,
        out_type=jax.ShapeDtypeStruct((batch, feat), jnp.float32),
        scratch_types=[
            pltpu.VMEM((b_per_w,), jnp.int32),
            pltpu.VMEM((b_per_w, feat), jnp.float32),
            pltpu.SemaphoreType.DMA,
        ],
    )
    def gather_kernel(table_hbm, idx_hbm, out_hbm, idx_v, rows_v, sem):
        wid = lax.axis_index("s") * info.num_cores + lax.axis_index("c")
        base = wid * b_per_w
        pltpu.sync_copy(idx_hbm.at[pl.ds(base, b_per_w)], idx_v)
        pltpu.async_copy(table_hbm.at[idx_v], rows_v, sem).wait()
        pltpu.sync_copy(rows_v, out_hbm.at[pl.ds(base, b_per_w)])

    return gather_kernel(matrix, target)


def _loss_kernel(m_ref, xf_ref, c_ref, out_ref, xn_s, cn_s):
    j = pl.program_id(0)
    batch = xf_ref.shape[0]

    @pl.when(j == 0)
    def _init():
        x = xf_ref[...]
        c = c_ref[...]
        xn_s[...] = (x * lax.rsqrt(jnp.sum(x * x, axis=1, keepdims=True))).astype(
            jnp.bfloat16
        )
        cn_s[...] = (c * lax.rsqrt(jnp.sum(c * c, axis=1, keepdims=True))).astype(
            jnp.bfloat16
        )
        icg = jnp.sum(jnp.square(x - c)) * (WEIGHT_CENT / batch)
        out_ref[...] = jnp.full((1, 1), icg, jnp.float32)

    m = m_ref[...]
    mn = (
        m * (lax.rsqrt(jnp.sum(m * m, axis=1, keepdims=True)) * (LOG2E / TAU))
    ).astype(jnp.bfloat16)
    dims = (((1,), (1,)), ((), ()))
    cn = cn_s[...]
    xn = xn_s[...]
    part = jnp.zeros((), jnp.float32)
    for t in range(TILE_N // SUB_N):
        mt = mn[t * SUB_N : (t + 1) * SUB_N, :]
        sa = lax.dot_general(cn, mt, dims, preferred_element_type=jnp.float32)
        sb = lax.dot_general(xn, mt, dims, preferred_element_type=jnp.float32)
        d = jnp.exp2(sa) - jnp.exp2(sb)
        part = part + jnp.sum(d * d)
    out_ref[...] = out_ref[...] + jnp.full((1, 1), part * (ETA / batch), jnp.float32)


def _tc_loss(xf, centers, matrix):
    batch, feat = xf.shape
    n_cls = matrix.shape[0]
    grid = (n_cls // TILE_N,)
    out = pl.pallas_call(
        _loss_kernel,
        grid=grid,
        in_specs=[
            pl.BlockSpec((TILE_N, feat), lambda j: (j, 0)),
            pl.BlockSpec((batch, feat), lambda j: (0, 0)),
            pl.BlockSpec((batch, feat), lambda j: (0, 0)),
        ],
        out_specs=pl.BlockSpec((1, 1), lambda j: (0, 0)),
        out_shape=jax.ShapeDtypeStruct((1, 1), jnp.float32),
        scratch_shapes=[
            pltpu.VMEM((batch, feat), jnp.bfloat16),
            pltpu.VMEM((batch, feat), jnp.bfloat16),
        ],
    )(matrix, xf, centers)
    return out[0, 0]


def kernel(xf, target, epoch, matrix):
    centers = _sc_gather(matrix, target.astype(jnp.int32))
    return _tc_loss(xf, centers, matrix)


# speedup vs baseline: 1.4806x; 1.0228x over previous
import functools

import jax
import jax.numpy as jnp
from jax import lax
from jax.experimental import pallas as pl
from jax.experimental.pallas import tpu as pltpu
from jax.experimental.pallas import tpu_sc as plsc

WEIGHT_CENT = 0.001
TAU = 1.0
ETA = 2.0
LOG2E = 1.4426950408889634

TILE_N = 1024
SUB_N = 256


def _sc_gather(matrix, target):
    n_cls, feat = matrix.shape
    batch = target.shape[0]
    info = plsc.get_sparse_core_info()
    nw = info.num_cores * info.num_subcores
    b_per_w = batch // nw
    mesh = plsc.VectorSubcoreMesh(core_axis_name="c", subcore_axis_name="s")

    @functools.partial(
        pl.kernel,
        mesh=mesh,
        out_type=jax.ShapeDtypeStruct((batch, feat), jnp.float32),
        scratch_types=[
            pltpu.VMEM((b_per_w,), jnp.int32),
            pltpu.VMEM((b_per_w, feat), jnp.float32),
            pltpu.SemaphoreType.DMA,
        ],
    )
    def gather_kernel(table_hbm, idx_hbm, out_hbm, idx_v, rows_v, sem):
        wid = lax.axis_index("s") * info.num_cores + lax.axis_index("c")
        base = wid * b_per_w
        pltpu.sync_copy(idx_hbm.at[pl.ds(base, b_per_w)], idx_v)
        pltpu.async_copy(table_hbm.at[idx_v], rows_v, sem).wait()
        pltpu.sync_copy(rows_v, out_hbm.at[pl.ds(base, b_per_w)])

    return gather_kernel(matrix, target)


def _loss_kernel(m_ref, xf_ref, c_ref, out_ref, xn_s, cn_s):
    j = pl.program_id(0)
    batch = xf_ref.shape[0]

    @pl.when(j == 0)
    def _init():
        x = xf_ref[...]
        c = c_ref[...]
        xn_s[...] = (x * lax.rsqrt(jnp.sum(x * x, axis=1, keepdims=True))).astype(
            jnp.bfloat16
        )
        cn_s[...] = (c * lax.rsqrt(jnp.sum(c * c, axis=1, keepdims=True))).astype(
            jnp.bfloat16
        )
        icg = jnp.sum(jnp.square(x - c)) * (WEIGHT_CENT / batch)
        out_ref[...] = jnp.full((1, 1), icg, jnp.float32)

    m = m_ref[...]
    mn = (
        m * (lax.rsqrt(jnp.sum(m * m, axis=1, keepdims=True)) * (LOG2E / TAU))
    ).astype(jnp.bfloat16)
    dims = (((1,), (1,)), ((), ()))
    cn = cn_s[...]
    xn = xn_s[...]
    part = jnp.zeros((), jnp.float32)
    for t in range(TILE_N // SUB_N):
        mt = mn[t * SUB_N : (t + 1) * SUB_N, :]
        sa = lax.dot_general(cn, mt, dims, preferred_element_type=jnp.float32)
        sb = lax.dot_general(xn, mt, dims, preferred_element_type=jnp.float32)
        d = jnp.exp2(sa) - jnp.exp2(sb)
        part = part + jnp.sum(d * d)
    out_ref[...] = out_ref[...] + jnp.full((1, 1), part * (ETA / batch), jnp.float32)


def _tc_loss(xf, centers, matrix):
    batch, feat = xf.shape
    n_cls = matrix.shape[0]
    grid = (n_cls // TILE_N,)
    out = pl.pallas_call(
        _loss_kernel,
        grid=grid,
        in_specs=[
            pl.BlockSpec((TILE_N, feat), lambda j: (j, 0)),
            pl.BlockSpec((batch, feat), lambda j: (0, 0)),
            pl.BlockSpec((batch, feat), lambda j: (0, 0)),
        ],
        out_specs=pl.BlockSpec((1, 1), lambda j: (0, 0)),
        out_shape=jax.ShapeDtypeStruct((1, 1), jnp.float32),
        scratch_shapes=[
            pltpu.VMEM((batch, feat), jnp.bfloat16),
            pltpu.VMEM((batch, feat), jnp.bfloat16),
        ],
    )(matrix, xf, centers)
    return out[0, 0]


def kernel(xf, target, epoch, matrix):
    centers = _sc_gather(matrix, target.astype(jnp.int32))
    return _tc_loss(xf, centers, matrix)


# speedup vs baseline: 1.5093x; 1.0194x over previous
import functools

import jax
import jax.numpy as jnp
from jax import lax
from jax.experimental import pallas as pl
from jax.experimental.pallas import tpu as pltpu
from jax.experimental.pallas import tpu_sc as plsc

WEIGHT_CENT = 0.001
TAU = 1.0
ETA = 2.0
LOG2E = 1.4426950408889634

TILE_N = 2048
SUB_N = 256


def _sc_gather(matrix, target):
    n_cls, feat = matrix.shape
    batch = target.shape[0]
    info = plsc.get_sparse_core_info()
    nw = info.num_cores * info.num_subcores
    b_per_w = batch // nw
    mesh = plsc.VectorSubcoreMesh(core_axis_name="c", subcore_axis_name="s")

    @functools.partial(
        pl.kernel,
        mesh=mesh,
        out_type=jax.ShapeDtypeStruct((batch, feat), jnp.float32),
        scratch_types=[
            pltpu.VMEM((b_per_w,), jnp.int32),
            pltpu.VMEM((b_per_w, feat), jnp.float32),
            pltpu.SemaphoreType.DMA,
        ],
    )
    def gather_kernel(table_hbm, idx_hbm, out_hbm, idx_v, rows_v, sem):
        wid = lax.axis_index("s") * info.num_cores + lax.axis_index("c")
        base = wid * b_per_w
        pltpu.sync_copy(idx_hbm.at[pl.ds(base, b_per_w)], idx_v)
        pltpu.async_copy(table_hbm.at[idx_v], rows_v, sem).wait()
        pltpu.sync_copy(rows_v, out_hbm.at[pl.ds(base, b_per_w)])

    return gather_kernel(matrix, target)


def _loss_kernel(m_ref, xf_ref, c_ref, out_ref, xn_s, cn_s):
    j = pl.program_id(0)
    batch = xf_ref.shape[0]

    @pl.when(j == 0)
    def _init():
        x = xf_ref[...]
        c = c_ref[...]
        xn_s[...] = (x * lax.rsqrt(jnp.sum(x * x, axis=1, keepdims=True))).astype(
            jnp.bfloat16
        )
        cn_s[...] = (c * lax.rsqrt(jnp.sum(c * c, axis=1, keepdims=True))).astype(
            jnp.bfloat16
        )
        icg = jnp.sum(jnp.square(x - c)) * (WEIGHT_CENT / batch)
        out_ref[...] = jnp.full((1, 1), icg, jnp.float32)

    m = m_ref[...]
    mn = (
        m * (lax.rsqrt(jnp.sum(m * m, axis=1, keepdims=True)) * (LOG2E / TAU))
    ).astype(jnp.bfloat16)
    dims = (((1,), (1,)), ((), ()))
    cn = cn_s[...]
    xn = xn_s[...]
    part = jnp.zeros((), jnp.float32)
    for t in range(TILE_N // SUB_N):
        mt = mn[t * SUB_N : (t + 1) * SUB_N, :]
        sa = lax.dot_general(cn, mt, dims, preferred_element_type=jnp.float32)
        sb = lax.dot_general(xn, mt, dims, preferred_element_type=jnp.float32)
        d = jnp.exp2(sa) - jnp.exp2(sb)
        part = part + jnp.sum(d * d)
    out_ref[...] = out_ref[...] + jnp.full((1, 1), part * (ETA / batch), jnp.float32)


def _tc_loss(xf, centers, matrix):
    batch, feat = xf.shape
    n_cls = matrix.shape[0]
    grid = (n_cls // TILE_N,)
    out = pl.pallas_call(
        _loss_kernel,
        grid=grid,
        in_specs=[
            pl.BlockSpec((TILE_N, feat), lambda j: (j, 0)),
            pl.BlockSpec((batch, feat), lambda j: (0, 0)),
            pl.BlockSpec((batch, feat), lambda j: (0, 0)),
        ],
        out_specs=pl.BlockSpec((1, 1), lambda j: (0, 0)),
        out_shape=jax.ShapeDtypeStruct((1, 1), jnp.float32),
        scratch_shapes=[
            pltpu.VMEM((batch, feat), jnp.bfloat16),
            pltpu.VMEM((batch, feat), jnp.bfloat16),
        ],
    )(matrix, xf, centers)
    return out[0, 0]


def kernel(xf, target, epoch, matrix):
    centers = _sc_gather(matrix, target.astype(jnp.int32))
    return _tc_loss(xf, centers, matrix)
